# baseline (speedup 1.0000x reference)
import functools

import jax
import jax.numpy as jnp
from jax import lax
from jax.experimental import pallas as pl
from jax.experimental.pallas import tpu as pltpu
from jax.experimental.pallas import tpu_sc as plsc

B = 4096
L = 200
DIM = 64
VOCAB = 100000
ROWS = B * L
PADDIM = 128

NC = 2
NS = 16
NW = NC * NS
ROWS_PER_W = ROWS // NW
CHUNK = 320
N_CHUNKS = ROWS_PER_W // CHUNK

_sc_mesh = plsc.VectorSubcoreMesh(core_axis_name="c", subcore_axis_name="s")


@functools.partial(
    pl.kernel,
    mesh=_sc_mesh,
    out_type=jax.ShapeDtypeStruct((ROWS, PADDIM), jnp.float32),
    scratch_types=[
        pltpu.VMEM((ROWS_PER_W,), jnp.int32),
        pltpu.VMEM((CHUNK, PADDIM), jnp.float32),
        pltpu.VMEM((CHUNK, PADDIM), jnp.float32),
        pltpu.SemaphoreType.DMA,
        pltpu.SemaphoreType.DMA,
        pltpu.SemaphoreType.DMA,
        pltpu.SemaphoreType.DMA,
    ],
)
def _sc_gather(idx_hbm, table_hbm, out_hbm, idx_v, rows0, rows1, sg0, sg1, so0, so1):
    wid = lax.axis_index("s") * NC + lax.axis_index("c")
    base = wid * ROWS_PER_W
    pltpu.sync_copy(idx_hbm.at[pl.ds(base, ROWS_PER_W)], idx_v)
    rows = (rows0, rows1)
    sg = (sg0, sg1)
    so = (so0, so1)

    def start_gather(i, b):
        pltpu.async_copy(
            table_hbm.at[idx_v.at[pl.ds(i * CHUNK, CHUNK)]], rows[b], sg[b]
        )

    def wait_gather(b):
        pltpu.make_async_copy(
            table_hbm.at[idx_v.at[pl.ds(0, CHUNK)]], rows[b], sg[b]
        ).wait()

    def start_out(i, b):
        pltpu.async_copy(rows[b], out_hbm.at[pl.ds(base + i * CHUNK, CHUNK)], so[b])

    def wait_out(b):
        pltpu.make_async_copy(
            rows[b], out_hbm.at[pl.ds(base, CHUNK)], so[b]
        ).wait()

    def body(t, carry):
        for bb in (0, 1):
            i = 2 * t + bb
            if bb == 0:
                @pl.when(t >= 1)
                def _():
                    wait_out(0)
            else:
                @pl.when(t >= 1)
                def _():
                    wait_out(1)
            start_gather(i, bb)
            if bb == 0:
                @pl.when(t >= 1)
                def _():
                    wait_gather(1)
                    start_out(i - 1, 1)
            else:
                wait_gather(0)
                start_out(i - 1, 0)
        return carry

    lax.fori_loop(0, N_CHUNKS // 2, body, 0)
    wait_gather(1)
    start_out(N_CHUNKS - 1, 1)
    wait_out(0)
    wait_out(1)


TBLK = 4000


def _pad_body(t_ref, o_ref):
    o_ref[...] = jnp.concatenate(
        [t_ref[...], jnp.zeros((TBLK, PADDIM - DIM), jnp.float32)], axis=1
    )


def _pad_table(table):
    return pl.pallas_call(
        _pad_body,
        grid=(VOCAB // TBLK,),
        in_specs=[pl.BlockSpec((TBLK, DIM), lambda i: (i, 0))],
        out_specs=pl.BlockSpec((TBLK, PADDIM), lambda i: (i, 0)),
        out_shape=jax.ShapeDtypeStruct((VOCAB, PADDIM), jnp.float32),
    )(table)


BLK = 4096


def _dense_body(e_ref, wc_ref, bc_ref, wa_ref, ba_ref, o_ref):
    e = e_ref[:, :DIM]
    h = jnp.tanh(
        jnp.dot(e, wc_ref[...], preferred_element_type=jnp.float32) + bc_ref[...]
    )
    o_ref[...] = (
        e + jnp.dot(h, wa_ref[...], preferred_element_type=jnp.float32) + ba_ref[...]
    )


def _dense(emb, wc, bc, wa, ba):
    return pl.pallas_call(
        _dense_body,
        grid=(ROWS // BLK,),
        in_specs=[
            pl.BlockSpec((BLK, PADDIM), lambda i: (i, 0)),
            pl.BlockSpec((DIM, 3 * DIM), lambda i: (0, 0)),
            pl.BlockSpec((1, 3 * DIM), lambda i: (0, 0)),
            pl.BlockSpec((3 * DIM, DIM), lambda i: (0, 0)),
            pl.BlockSpec((1, DIM), lambda i: (0, 0)),
        ],
        out_specs=pl.BlockSpec((BLK, DIM), lambda i: (i, 0)),
        out_shape=jax.ShapeDtypeStruct((ROWS, DIM), jnp.float32),
    )(emb, wc, bc, wa, ba)


def kernel(phoneme_input, table, W_ph, b_ph, W_em, b_em, W_vo, b_vo, W_al, b_al):
    idx = phoneme_input.reshape(-1).astype(jnp.int32)
    table_p = _pad_table(table)
    emb = _sc_gather(idx, table_p)
    wc = jnp.concatenate([W_ph.T, W_em.T, W_vo.T], axis=1)
    bc = jnp.concatenate([b_ph, b_em, b_vo]).reshape(1, 3 * DIM)
    wa = W_al.T
    ba = b_al.reshape(1, DIM)
    out = _dense(emb, wc, bc, wa, ba)
    return out.reshape(B, L, DIM)


# speedup vs baseline: 1.3582x; 1.3582x over previous
import functools

import jax
import jax.numpy as jnp
from jax import lax
from jax.experimental import pallas as pl
from jax.experimental.pallas import tpu as pltpu
from jax.experimental.pallas import tpu_sc as plsc

B = 4096
L = 200
DIM = 64
VOCAB = 100000
ROWS = B * L
PADDIM = 128

NC = 2
NS = 16
NW = NC * NS
ROWS_PER_W = ROWS // NW
CHUNK = 320
N_CHUNKS = ROWS_PER_W // CHUNK

_sc_mesh = plsc.VectorSubcoreMesh(core_axis_name="c", subcore_axis_name="s")


@functools.partial(
    pl.kernel,
    mesh=_sc_mesh,
    out_type=jax.ShapeDtypeStruct((ROWS, PADDIM), jnp.float32),
    scratch_types=[
        pltpu.VMEM((ROWS_PER_W,), jnp.int32),
        pltpu.VMEM((CHUNK, PADDIM), jnp.float32),
        pltpu.VMEM((CHUNK, PADDIM), jnp.float32),
        pltpu.SemaphoreType.DMA,
        pltpu.SemaphoreType.DMA,
        pltpu.SemaphoreType.DMA,
        pltpu.SemaphoreType.DMA,
    ],
)
def _sc_gather(idx_hbm, table_hbm, out_hbm, idx_v, rows0, rows1, sg0, sg1, so0, so1):
    wid = lax.axis_index("s") * NC + lax.axis_index("c")
    base = wid * ROWS_PER_W
    pltpu.sync_copy(idx_hbm.at[pl.ds(base, ROWS_PER_W)], idx_v)
    rows = (rows0, rows1)
    sg = (sg0, sg1)
    so = (so0, so1)

    def start_gather(i, b):
        pltpu.async_copy(
            table_hbm.at[idx_v.at[pl.ds(i * CHUNK, CHUNK)]], rows[b], sg[b]
        )

    def wait_gather(b):
        pltpu.make_async_copy(
            table_hbm.at[idx_v.at[pl.ds(0, CHUNK)]], rows[b], sg[b]
        ).wait()

    def start_out(i, b):
        pltpu.async_copy(rows[b], out_hbm.at[pl.ds(base + i * CHUNK, CHUNK)], so[b])

    def wait_out(b):
        pltpu.make_async_copy(
            rows[b], out_hbm.at[pl.ds(base, CHUNK)], so[b]
        ).wait()

    def body(t, carry):
        for bb in (0, 1):
            i = 2 * t + bb
            if bb == 0:
                @pl.when(t >= 1)
                def _():
                    wait_out(0)
            else:
                @pl.when(t >= 1)
                def _():
                    wait_out(1)
            start_gather(i, bb)
            if bb == 0:
                @pl.when(t >= 1)
                def _():
                    wait_gather(1)
                    start_out(i - 1, 1)
            else:
                wait_gather(0)
                start_out(i - 1, 0)
        return carry

    lax.fori_loop(0, N_CHUNKS // 2, body, 0)
    wait_gather(1)
    start_out(N_CHUNKS - 1, 1)
    wait_out(0)
    wait_out(1)


TBLK = 4000


def _pad_body(t_ref, o_ref):
    o_ref[...] = jnp.concatenate(
        [t_ref[...], jnp.zeros((TBLK, PADDIM - DIM), jnp.float32)], axis=1
    )


def _pad_table(table):
    return pl.pallas_call(
        _pad_body,
        grid=(VOCAB // TBLK,),
        in_specs=[pl.BlockSpec((TBLK, DIM), lambda i: (i, 0))],
        out_specs=pl.BlockSpec((TBLK, PADDIM), lambda i: (i, 0)),
        out_shape=jax.ShapeDtypeStruct((VOCAB, PADDIM), jnp.float32),
    )(table)


BLKB = 4096


def _dense_body(e_ref, wq_ref, bc_ref, wa_ref, ba_ref, o_ref):
    e = e_ref[0, :, :DIM]
    g = jax.lax.dot_general(
        wq_ref[...], e, (((1,), (1,)), ((), ())),
        preferred_element_type=jnp.float32,
    )
    h = jnp.tanh(g[: 3 * DIM] + bc_ref[...])
    o_ref[0] = (
        g[3 * DIM :]
        + jnp.dot(wa_ref[...], h, preferred_element_type=jnp.float32)
        + ba_ref[...]
    )


def _dense(emb3, wq, bc_col, wa, ba_col):
    return pl.pallas_call(
        _dense_body,
        grid=(L, B // BLKB),
        in_specs=[
            pl.BlockSpec((1, BLKB, PADDIM), lambda l, j: (l, j, 0)),
            pl.BlockSpec((4 * DIM, DIM), lambda l, j: (0, 0)),
            pl.BlockSpec((3 * DIM, 1), lambda l, j: (0, 0)),
            pl.BlockSpec((DIM, 3 * DIM), lambda l, j: (0, 0)),
            pl.BlockSpec((DIM, 1), lambda l, j: (0, 0)),
        ],
        out_specs=pl.BlockSpec((1, DIM, BLKB), lambda l, j: (l, 0, j)),
        out_shape=jax.ShapeDtypeStruct((L, DIM, B), jnp.float32),
    )(emb3, wq, bc_col, wa, ba_col)


def kernel(phoneme_input, table, W_ph, b_ph, W_em, b_em, W_vo, b_vo, W_al, b_al):
    idx_t = phoneme_input.T.reshape(-1).astype(jnp.int32)
    table_p = _pad_table(table)
    emb = _sc_gather(idx_t, table_p)
    emb3 = emb.reshape(L, B, PADDIM)
    wq = jnp.concatenate([W_ph, W_em, W_vo, jnp.eye(DIM, dtype=jnp.float32)], axis=0)
    bc_col = jnp.concatenate([b_ph, b_em, b_vo]).reshape(3 * DIM, 1)
    wa = W_al
    ba_col = b_al.reshape(DIM, 1)
    out_t = _dense(emb3, wq, bc_col, wa, ba_col)
    return out_t.transpose(2, 0, 1)
